# Initial kernel scaffold; baseline (speedup 1.0000x reference)
import jax
import jax.numpy as jnp
from jax.experimental import pallas as pl


def kernel(x, token_table, position_table):
    raise NotImplementedError("write your pallas kernel here")


# speedup vs baseline: 2.6042x; 2.6042x over previous
import functools

import jax
import jax.numpy as jnp
from jax import lax
from jax.experimental import pallas as pl
from jax.experimental.pallas import tpu as pltpu
from jax.experimental.pallas import tpu_sc as plsc

B = 1024
L = 200
D = 64
VOCAB = 100000

NUM_CORES = 2
NUM_SUBCORES = 16
NW = NUM_CORES * NUM_SUBCORES
SEQ_PER_W = B // NW

SPLIT = 104

_mesh = plsc.VectorSubcoreMesh(core_axis_name="c", subcore_axis_name="s")


@functools.partial(
    pl.kernel,
    out_type=jax.ShapeDtypeStruct((B, L, D), jnp.float32),
    mesh=_mesh,
    scratch_types=[
        pltpu.VMEM((L,), jnp.int32),
        pltpu.VMEM((L, D), jnp.float32),
        pltpu.VMEM((L, D), jnp.float32),
        pltpu.SemaphoreType.DMA,
    ],
    compiler_params=pltpu.CompilerParams(use_tc_tiling_on_sc=False),
)
def _embed_kernel(x_hbm, tok_hbm, pos_hbm, out_hbm, idx_v, rows_v, pos_v, sem):
    wid = lax.axis_index("s") * NUM_CORES + lax.axis_index("c")
    pltpu.sync_copy(pos_hbm, pos_v)

    @pl.loop(0, SEQ_PER_W)
    def _seq(i):
        s = wid * SEQ_PER_W + i
        pltpu.sync_copy(x_hbm.at[s], idx_v)
        cp1 = pltpu.async_copy(
            tok_hbm.at[idx_v.at[pl.ds(0, SPLIT)]],
            rows_v.at[pl.ds(0, SPLIT)], sem)
        cp2 = pltpu.async_copy(
            tok_hbm.at[idx_v.at[pl.ds(SPLIT, L - SPLIT)]],
            rows_v.at[pl.ds(SPLIT, L - SPLIT)], sem)
        cp1.wait()
        cp2.wait()

        @plsc.parallel_loop(0, L, unroll=4)
        def _row(r):
            for c in range(D // 16):
                sl = pl.ds(c * 16, 16)
                rows_v[r, sl] = rows_v[r, sl] + pos_v[r, sl]

        pltpu.sync_copy(rows_v, out_hbm.at[s])


def kernel(x, token_table, position_table):
    return _embed_kernel(x, token_table, position_table)


# speedup vs baseline: 3.1977x; 1.2279x over previous
import functools

import jax
import jax.numpy as jnp
from jax import lax
from jax.experimental import pallas as pl
from jax.experimental.pallas import tpu as pltpu
from jax.experimental.pallas import tpu_sc as plsc

B = 1024
L = 200
D = 64
VOCAB = 100000

NUM_CORES = 2
NUM_SUBCORES = 16
NW = NUM_CORES * NUM_SUBCORES
ROWS_PER_W = B * L // NW
CHUNK = 128
NCH = ROWS_PER_W // CHUNK
NBUF = 5
LOOKAHEAD = 2

_mesh = plsc.VectorSubcoreMesh(core_axis_name="c", subcore_axis_name="s")


@functools.partial(
    pl.kernel,
    out_type=jax.ShapeDtypeStruct((B * L, D), jnp.float32),
    mesh=_mesh,
    scratch_types=[
        pltpu.VMEM((ROWS_PER_W,), jnp.int32),
        pltpu.VMEM((NBUF, CHUNK, D), jnp.float32),
        pltpu.VMEM((2 * L, D), jnp.float32),
        pltpu.SemaphoreType.DMA((NBUF,)),
        pltpu.SemaphoreType.DMA((NBUF,)),
    ],
    compiler_params=pltpu.CompilerParams(use_tc_tiling_on_sc=False),
)
def _embed_kernel(x_hbm, tok_hbm, pos_hbm, out_hbm,
                  idx_all, rows, pos2, sem_g, sem_o):
    wid = lax.axis_index("s") * NUM_CORES + lax.axis_index("c")
    wbase = wid * ROWS_PER_W

    pltpu.sync_copy(x_hbm.at[wid], idx_all)
    pltpu.sync_copy(pos_hbm, pos2.at[pl.ds(0, L)])
    pltpu.sync_copy(pos_hbm, pos2.at[pl.ds(L, L)])

    def g_issue(i, b):
        pltpu.async_copy(
            tok_hbm.at[idx_all.at[pl.ds(i * CHUNK, CHUNK)]],
            rows.at[b], sem_g.at[b])

    def g_wait(i, b):
        pltpu.make_async_copy(
            tok_hbm.at[idx_all.at[pl.ds(i * CHUNK, CHUNK)]],
            rows.at[b], sem_g.at[b]).wait()

    def o_issue(i, b):
        pltpu.async_copy(
            rows.at[b], out_hbm.at[pl.ds(wbase + i * CHUNK, CHUNK)],
            sem_o.at[b])

    def o_wait(i, b):
        pltpu.make_async_copy(
            rows.at[b], out_hbm.at[pl.ds(wbase + i * CHUNK, CHUNK)],
            sem_o.at[b]).wait()

    def chunk_step(i, b, issue_next, out_wait):
        g_wait(i, b)
        b2 = (b + LOOKAHEAD) % NBUF
        if out_wait:
            o_wait(i - (NBUF - LOOKAHEAD), b2)
        if issue_next:
            g_issue(i + LOOKAHEAD, b2)
        p = lax.rem(i * CHUNK, L)
        rows_b = rows.at[b]

        @plsc.parallel_loop(0, CHUNK, unroll=4)
        def _row(r):
            for c in range(D // 16):
                sl = pl.ds(c * 16, 16)
                rows_b[r, sl] = rows_b[r, sl] + pos2[p + r, sl]

        o_issue(i, b)

    for j in range(LOOKAHEAD):
        g_issue(j, j)
    for i in range(NBUF):
        chunk_step(i, i, True, i >= NBUF - LOOKAHEAD)

    @pl.loop(1, NCH // NBUF - 1)
    def _group(g):
        i0 = g * NBUF
        for b in range(NBUF):
            chunk_step(i0 + b, b, True, True)

    for i in range(NCH - NBUF, NCH):
        chunk_step(i, i % NBUF, i + LOOKAHEAD < NCH, True)
    for i in range(NCH - NBUF + LOOKAHEAD, NCH):
        o_wait(i, i % NBUF)


def kernel(x, token_table, position_table):
    out = _embed_kernel(x.reshape(NW, ROWS_PER_W), token_table, position_table)
    return out.reshape(B, L, D)


# speedup vs baseline: 4.6232x; 1.4458x over previous
import functools

import jax
import jax.numpy as jnp
from jax import lax
from jax.experimental import pallas as pl
from jax.experimental.pallas import tpu as pltpu
from jax.experimental.pallas import tpu_sc as plsc

B = 1024
L = 200
D = 64
DPAD = 128
VOCAB = 100000

NUM_CORES = 2
NUM_SUBCORES = 16
NW = NUM_CORES * NUM_SUBCORES
SEQ_W = B // NW
HALF0 = 104
HALF1 = L - HALF0
NCHUNK = 2 * SEQ_W
NBUF = 4
LOOKAHEAD = 2

_mesh = plsc.VectorSubcoreMesh(core_axis_name="c", subcore_axis_name="s")


@functools.partial(
    pl.kernel,
    out_type=jax.ShapeDtypeStruct((B, L, DPAD), jnp.float32),
    mesh=_mesh,
    scratch_types=[
        pltpu.VMEM((SEQ_W, L), jnp.int32),
        pltpu.VMEM((NBUF, HALF0, D), jnp.float32),
        pltpu.VMEM((L, D), jnp.float32),
        pltpu.SemaphoreType.DMA((NBUF,)),
        pltpu.SemaphoreType.DMA((NBUF,)),
    ],
    compiler_params=pltpu.CompilerParams(use_tc_tiling_on_sc=False),
)
def _embed_kernel(x_hbm, tok_hbm, pos_hbm, out_hbm,
                  idx2, rows, pos_v, sem_g, sem_o):
    wid = lax.axis_index("s") * NUM_CORES + lax.axis_index("c")
    sbase = wid * SEQ_W

    pltpu.sync_copy(x_hbm.at[pl.ds(sbase, SEQ_W)], idx2)
    pltpu.sync_copy(pos_hbm, pos_v)

    def halves(k, b):
        h = b & 1
        off = HALF0 * h
        n = HALF1 if h else HALF0
        return k >> 1, off, n

    def g_issue(k, b):
        s, off, n = halves(k, b)
        pltpu.async_copy(
            tok_hbm.at[idx2.at[s, pl.ds(off, n)]],
            rows.at[b, pl.ds(0, n)], sem_g.at[b])

    def g_wait(k, b):
        s, off, n = halves(k, b)
        pltpu.make_async_copy(
            tok_hbm.at[idx2.at[s, pl.ds(off, n)]],
            rows.at[b, pl.ds(0, n)], sem_g.at[b]).wait()

    def o_copy(k, b):
        s, off, n = halves(k, b)
        return pltpu.make_async_copy(
            rows.at[b, pl.ds(0, n)],
            out_hbm.at[sbase + s, pl.ds(off, n), pl.ds(0, D)],
            sem_o.at[b])

    def chunk_step(k, b, issue_next, out_wait):
        g_wait(k, b)
        b2 = (b + LOOKAHEAD) % NBUF
        if out_wait:
            o_copy(k - (NBUF - LOOKAHEAD), b2).wait()
        if issue_next:
            g_issue(k + LOOKAHEAD, b2)
        _, off, n = halves(k, b)
        rows_b = rows.at[b]

        @plsc.parallel_loop(0, n, unroll=4)
        def _row(r):
            for c in range(D // 16):
                sl = pl.ds(c * 16, 16)
                rows_b[r, sl] = rows_b[r, sl] + pos_v[off + r, sl]

        o_copy(k, b).start()

    for j in range(LOOKAHEAD):
        g_issue(j, j)
    for k in range(NBUF):
        chunk_step(k, k, True, k >= NBUF - LOOKAHEAD)

    @pl.loop(1, NCHUNK // NBUF - 1)
    def _group(g):
        k0 = g * NBUF
        for b in range(NBUF):
            chunk_step(k0 + b, b, True, True)

    for k in range(NCHUNK - NBUF, NCHUNK):
        chunk_step(k, k % NBUF, k + LOOKAHEAD < NCHUNK, True)
    for k in range(NCHUNK - NBUF + LOOKAHEAD, NCHUNK):
        o_copy(k, k % NBUF).wait()


def kernel(x, token_table, position_table):
    out = _embed_kernel(x, token_table, position_table)
    return out[:, :, :D]
